# baseline (speedup 1.0000x reference)
import functools

import numpy as np

import jax
import jax.numpy as jnp
from jax import lax
from jax.experimental import pallas as pl
from jax.experimental.pallas import tpu as pltpu
from jax.experimental.pallas import tpu_sc as plsc

_OPTIMAL_SCALE = 2.513930578568423
_N_LEVELS = 16
_MAGIC = np.float32(1.5 * 2**23)

_ROW = 4096
_CR = 4
_CHUNK = _CR * _ROW
_NW = 32
_L = 16


def _newton_rsqrt(m):
    i = plsc.bitcast(m, jnp.int32)
    i = jnp.int32(0x5F3759DF) - lax.shift_right_arithmetic(i, 1)
    y = plsc.bitcast(i, jnp.float32)
    for _ in range(4):
        y = y * (jnp.float32(1.5) - jnp.float32(0.5) * m * y * y)
    return y


def _sc_body(x_hbm, out_hbm, xbufs, obufs, psum_row, in_sems, out_sems):
    wid = lax.axis_index("c") * 16 + lax.axis_index("s")
    rows_per_w = 8192 // _NW
    chunks = rows_per_w // _CR
    base_row = wid * rows_per_w

    iot = lax.iota(jnp.int32, _L)
    perm1 = iot ^ 1
    perm2 = iot ^ 2
    perm4 = iot ^ 4
    perm6 = iot ^ 6
    tb_b = lax.shift_right_logical(iot, 1) & 1
    tb_cd = lax.shift_right_logical(iot, 2) & 1

    def chunk_start(c):
        return base_row + c * _CR

    def start_in(c, b):
        pltpu.async_copy(x_hbm.at[pl.ds(chunk_start(c), _CR)], xbufs[b],
                         in_sems[b])

    def wait_in(c, b):
        pltpu.make_async_copy(x_hbm.at[pl.ds(chunk_start(c), _CR)],
                              xbufs[b], in_sems[b]).wait()

    def start_out(c, b):
        pltpu.async_copy(obufs[b], out_hbm.at[pl.ds(chunk_start(c), _CR)],
                         out_sems[b])

    def wait_out(c, b):
        pltpu.make_async_copy(obufs[b],
                              out_hbm.at[pl.ds(chunk_start(c), _CR)],
                              out_sems[b]).wait()

    def compute_row(r, b):
        xbuf = xbufs[b]
        obuf = obufs[b]

        z = jnp.zeros((_L,), jnp.float32)

        @plsc.parallel_loop(0, 256, 4, unroll=2, carry=(z, z, z, z))
        def pa_loop(j, accs):
            acc = list(accs)
            for u in range(4):
                q = j + u
                off = q * _L
                v = xbuf[r, pl.ds(off, _L)]
                p = plsc.load_gather(xbuf.at[r, pl.ds(off, _L)], [perm1])
                s = v * v
                psum_row[pl.ds(off, _L)] = s + p * p
                acc[u] = acc[u] + s
            return tuple(acc)

        a0, a1, a2, a3 = pa_loop
        total = jnp.sum((a0 + a1) + (a2 + a3))
        mean = jnp.full((_L,), total * jnp.float32(1.0 / _ROW))
        sqrt_m = mean * _newton_rsqrt(mean)
        scale = jnp.float32(_OPTIMAL_SCALE) * sqrt_m + jnp.float32(1e-8)
        step = scale * jnp.float32(2.0 / (_N_LEVELS - 1))
        halfstep = scale * jnp.float32(1.0 / (_N_LEVELS - 1))
        recip = jnp.float32(1.0) / step
        nscale = -scale

        @plsc.parallel_loop(0, 256, unroll=12)
        def pb_loop(q):
            off = q * _L
            v = xbuf[r, pl.ds(off, _L)]
            pslice = psum_row.at[pl.ds(off, _L)]
            k = plsc.bitcast(psum_row[pl.ds(off, _L)], jnp.int32)
            kb = plsc.bitcast(plsc.load_gather(pslice, [perm2]), jnp.int32)
            kc = plsc.bitcast(plsc.load_gather(pslice, [perm4]), jnp.int32)
            kd = plsc.bitcast(plsc.load_gather(pslice, [perm6]), jnp.int32)
            mb = kb - tb_b
            mc = kc - tb_cd
            md = kd - tb_cd
            med = jnp.maximum(jnp.minimum(mb, mc),
                              jnp.minimum(jnp.maximum(mb, mc), md))
            keep = med < k
            xc = jnp.minimum(jnp.maximum(v, nscale), scale)
            t = xc * recip + jnp.float32(0.5)
            rr = (t + _MAGIC) - _MAGIC
            xq = rr * step - halfstep
            obuf[r, pl.ds(off, _L)] = jnp.where(keep, xq, jnp.float32(0.0))

    start_in(0, 0)
    start_in(1, 1)

    def pair_body(i, carry):
        for b in (0, 1):
            c = 2 * i + b
            wait_in(c, b)

            @pl.when(i >= 1)
            def _():
                wait_out(c - 2, b)

            def row_body(r, carry2):
                compute_row(r, b)
                return carry2

            lax.fori_loop(0, _CR, row_body, 0)
            start_out(c, b)

            @pl.when(i <= chunks // 2 - 2)
            def _():
                start_in(c + 2, b)

        return carry

    lax.fori_loop(0, chunks // 2, pair_body, 0)
    wait_out(chunks - 2, 0)
    wait_out(chunks - 1, 1)


@jax.jit
def _sc_quantize(xf):
    mesh = plsc.VectorSubcoreMesh(core_axis_name="c", subcore_axis_name="s")
    body = functools.partial(
        pl.kernel,
        out_type=jax.ShapeDtypeStruct(xf.shape, jnp.float32),
        mesh=mesh,
        compiler_params=pltpu.CompilerParams(needs_layout_passes=False),
        scratch_types=[
            [pltpu.VMEM((_CR, _ROW), jnp.float32) for _ in range(2)],
            [pltpu.VMEM((_CR, _ROW), jnp.float32) for _ in range(2)],
            pltpu.VMEM((_ROW,), jnp.float32),
            [pltpu.SemaphoreType.DMA for _ in range(2)],
            [pltpu.SemaphoreType.DMA for _ in range(2)],
        ],
    )(_sc_body)
    return body(xf)


def kernel(x):
    shape = x.shape
    out = _sc_quantize(x.reshape(-1, shape[-1]))
    return out.reshape(shape)


# speedup vs baseline: 1.2442x; 1.2442x over previous
import functools

import numpy as np

import jax
import jax.numpy as jnp
from jax import lax
from jax.experimental import pallas as pl
from jax.experimental.pallas import tpu as pltpu
from jax.experimental.pallas import tpu_sc as plsc

_OPTIMAL_SCALE = 2.513930578568423
_N_LEVELS = 16
_MAGIC = np.float32(1.5 * 2**23)

_ROW = 4096
_CR = 4
_CHUNK = _CR * _ROW
_NW = 32
_L = 16


def _newton_rsqrt(m):
    i = plsc.bitcast(m, jnp.int32)
    i = jnp.int32(0x5F3759DF) - lax.shift_right_arithmetic(i, 1)
    y = plsc.bitcast(i, jnp.float32)
    for _ in range(4):
        y = y * (jnp.float32(1.5) - jnp.float32(0.5) * m * y * y)
    return y


def _sc_body(x_hbm, out_hbm, xbufs, obufs, psum_row, in_sems, out_sems):
    wid = lax.axis_index("c") * 16 + lax.axis_index("s")
    rows_per_w = 8192 // _NW
    chunks = rows_per_w // _CR
    base_row = wid * rows_per_w

    iot = lax.iota(jnp.int32, _L)
    perm1 = iot ^ 1
    perm2 = iot ^ 2
    perm4 = iot ^ 4
    perm6 = iot ^ 6
    tb_b = lax.shift_right_logical(iot, 1) & 1
    tb_cd = lax.shift_right_logical(iot, 2) & 1

    def chunk_start(c):
        return base_row + c * _CR

    def start_in(c, b):
        pltpu.async_copy(x_hbm.at[pl.ds(chunk_start(c), _CR)], xbufs[b],
                         in_sems[b])

    def wait_in(c, b):
        pltpu.make_async_copy(x_hbm.at[pl.ds(chunk_start(c), _CR)],
                              xbufs[b], in_sems[b]).wait()

    def start_out(c, b):
        pltpu.async_copy(obufs[b], out_hbm.at[pl.ds(chunk_start(c), _CR)],
                         out_sems[b])

    def wait_out(c, b):
        pltpu.make_async_copy(obufs[b],
                              out_hbm.at[pl.ds(chunk_start(c), _CR)],
                              out_sems[b]).wait()

    def compute_row(r, b):
        xbuf = xbufs[b]
        obuf = obufs[b]

        z = jnp.zeros((_L,), jnp.float32)

        @plsc.parallel_loop(0, 256, 4, unroll=2, carry=(z, z, z, z))
        def pa_loop(j, accs):
            acc = list(accs)
            for u in range(4):
                q = j + u
                off = q * _L
                v = xbuf[r, pl.ds(off, _L)]
                p = plsc.load_gather(xbuf.at[r, pl.ds(off, _L)], [perm1])
                s = v * v
                psum_row[pl.ds(off, _L)] = s + p * p
                acc[u] = acc[u] + s
            return tuple(acc)

        a0, a1, a2, a3 = pa_loop
        total = jnp.sum((a0 + a1) + (a2 + a3))
        mean = jnp.full((_L,), total * jnp.float32(1.0 / _ROW))
        sqrt_m = mean * _newton_rsqrt(mean)
        scale = jnp.float32(_OPTIMAL_SCALE) * sqrt_m + jnp.float32(1e-8)
        step = scale * jnp.float32(2.0 / (_N_LEVELS - 1))
        halfstep = scale * jnp.float32(1.0 / (_N_LEVELS - 1))
        recip = jnp.float32(1.0) / step
        nscale = -scale

        @plsc.parallel_loop(0, 256, unroll=8)
        def pb_loop(q):
            off = q * _L
            v = xbuf[r, pl.ds(off, _L)]
            pslice = psum_row.at[pl.ds(off, _L)]
            k = plsc.bitcast(psum_row[pl.ds(off, _L)], jnp.int32)
            kb = plsc.bitcast(plsc.load_gather(pslice, [perm2]), jnp.int32)
            kc = plsc.bitcast(plsc.load_gather(pslice, [perm4]), jnp.int32)
            kd = plsc.bitcast(plsc.load_gather(pslice, [perm6]), jnp.int32)
            mb = kb - tb_b
            mc = kc - tb_cd
            md = kd - tb_cd
            med = jnp.maximum(jnp.minimum(mb, mc),
                              jnp.minimum(jnp.maximum(mb, mc), md))
            keep = med < k
            xc = jnp.minimum(jnp.maximum(v, nscale), scale)
            t = xc * recip + jnp.float32(0.5)
            rr = (t + _MAGIC) - _MAGIC
            xq = rr * step - halfstep
            obuf[r, pl.ds(off, _L)] = jnp.where(keep, xq, jnp.float32(0.0))

    start_in(0, 0)
    start_in(1, 1)

    def pair_body(i, carry):
        for b in (0, 1):
            c = 2 * i + b
            wait_in(c, b)

            @pl.when(i >= 1)
            def _():
                wait_out(c - 2, b)

            def row_body(r, carry2):
                compute_row(r, b)
                return carry2

            lax.fori_loop(0, _CR, row_body, 0)
            start_out(c, b)

            @pl.when(i <= chunks // 2 - 2)
            def _():
                start_in(c + 2, b)

        return carry

    lax.fori_loop(0, chunks // 2, pair_body, 0)
    wait_out(chunks - 2, 0)
    wait_out(chunks - 1, 1)


@jax.jit
def _sc_quantize(xf):
    mesh = plsc.VectorSubcoreMesh(core_axis_name="c", subcore_axis_name="s")
    body = functools.partial(
        pl.kernel,
        out_type=jax.ShapeDtypeStruct(xf.shape, jnp.float32),
        mesh=mesh,
        compiler_params=pltpu.CompilerParams(needs_layout_passes=False),
        scratch_types=[
            [pltpu.VMEM((_CR, _ROW), jnp.float32) for _ in range(2)],
            [pltpu.VMEM((_CR, _ROW), jnp.float32) for _ in range(2)],
            pltpu.VMEM((_ROW,), jnp.float32),
            [pltpu.SemaphoreType.DMA for _ in range(2)],
            [pltpu.SemaphoreType.DMA for _ in range(2)],
        ],
    )(_sc_body)
    return body(xf)


def kernel(x):
    shape = x.shape
    out = _sc_quantize(x.reshape(-1, shape[-1]))
    return out.reshape(shape)
